# baseline (speedup 1.0000x reference)
import functools

import jax
import jax.numpy as jnp
from jax import lax
from jax.experimental import pallas as pl
from jax.experimental.pallas import tpu as pltpu
from jax.experimental.pallas import tpu_sc as plsc

N = 10000
E = 320000
D = 128
G = 128
NPAD = 10240
EPAD = 327680
HALF = 64
NS = 16
RPT = NPAD // NS
CH = 128
NROW = EPAD // CH
BLK = 10240
NB = NPAD // BLK

_mesh = lambda: plsc.VectorSubcoreMesh(core_axis_name="c", subcore_axis_name="s")
_sc_params = pltpu.CompilerParams(use_tc_tiling_on_sc=False)


def _degree_parts(dst2d, z1d):
    rpt_e = NROW // (2 * NS)

    @functools.partial(
        pl.kernel,
        out_type=jax.ShapeDtypeStruct((2, 1, NPAD), jnp.float32),
        mesh=_mesh(),
        compiler_params=_sc_params,
        scratch_types=[
            pltpu.VMEM((rpt_e, CH), jnp.int32),
            pltpu.VMEM((CH,), jnp.float32),
            pltpu.VMEM_SHARED((NPAD,), jnp.float32),
            pltpu.SemaphoreType.DMA,
        ],
    )
    def k(dst_hbm, z_hbm, out_hbm, dst_v, ones_v, hist_sh, sem):
        c = lax.axis_index("c")
        s = lax.axis_index("s")
        w = c * NS + s
        pltpu.sync_copy(z_hbm.at[pl.ds(s * RPT, RPT)], hist_sh.at[pl.ds(s * RPT, RPT)])
        pltpu.sync_copy(dst_hbm.at[pl.ds(w * rpt_e, rpt_e)], dst_v)
        for kk in range(CH // 16):
            ones_v[pl.ds(kk * 16, 16)] = jnp.ones((16,), jnp.float32)
        plsc.subcore_barrier()

        def body(i, carry):
            j = 8 * i
            for k in range(8):
                pltpu.async_copy(ones_v, hist_sh.at[dst_v.at[j + k]], sem, add=True)
            for k in range(8):
                pltpu.make_async_copy(ones_v, hist_sh.at[dst_v.at[j + k]], sem).wait()
            return carry

        lax.fori_loop(0, rpt_e // 8, body, 0)
        plsc.subcore_barrier()
        pltpu.sync_copy(hist_sh.at[pl.ds(s * RPT, RPT)],
                        out_hbm.at[c, 0, pl.ds(s * RPT, RPT)])

    return k(dst2d, z1d)


def _edge_scatter(a, srcx, dst2d, z2d):
    rpt_e = NROW // NS

    @functools.partial(
        pl.kernel,
        out_type=jax.ShapeDtypeStruct((2, NPAD, HALF), jnp.bfloat16),
        mesh=_mesh(),
        compiler_params=_sc_params,
        scratch_types=[
            pltpu.VMEM((rpt_e, CH), jnp.int32),
            pltpu.VMEM((rpt_e, CH), jnp.int32),
            [pltpu.VMEM((CH, HALF), jnp.bfloat16) for _ in range(8)],
            pltpu.VMEM_SHARED((NPAD, HALF), jnp.bfloat16),
            [pltpu.SemaphoreType.DMA for _ in range(8)],
            [pltpu.SemaphoreType.DMA for _ in range(8)],
            pltpu.SemaphoreType.DMA,
        ],
    )
    def k(a_hbm, src_hbm, dst_hbm, z_hbm, out_hbm,
          src_v, dst_v, msgs, s_sh, gsems, ssems, psem):
        NBUF = 8
        c = lax.axis_index("c")
        s = lax.axis_index("s")
        rows = pl.ds(s * RPT, RPT)
        pltpu.async_copy(src_hbm.at[c, pl.ds(s * rpt_e, rpt_e)], src_v, psem)
        pltpu.async_copy(dst_hbm.at[pl.ds(s * rpt_e, rpt_e)], dst_v, psem)
        pltpu.async_copy(z_hbm.at[rows], s_sh.at[rows], psem)
        pltpu.make_async_copy(src_hbm.at[c, pl.ds(s * rpt_e, rpt_e)], src_v, psem).wait()
        pltpu.make_async_copy(dst_hbm.at[pl.ds(s * rpt_e, rpt_e)], dst_v, psem).wait()
        pltpu.make_async_copy(z_hbm.at[rows], s_sh.at[rows], psem).wait()
        plsc.subcore_barrier()

        def gstart(j, k):
            pltpu.async_copy(a_hbm.at[src_v.at[j]], msgs[k], gsems[k])

        def gwait(j, k):
            pltpu.make_async_copy(a_hbm.at[src_v.at[j]], msgs[k], gsems[k]).wait()

        def sstart(j, k):
            pltpu.async_copy(msgs[k], s_sh.at[dst_v.at[j]], ssems[k], add=True)

        def swait(j, k):
            pltpu.make_async_copy(msgs[k], s_sh.at[dst_v.at[j]], ssems[k]).wait()

        for k in range(NBUF):
            gstart(k, k)

        def body(i, carry):
            j = NBUF * i
            for k in range(NBUF):
                gwait(j + k, k)
                sstart(j + k, k)
            for k in range(NBUF):
                swait(j + k, k)
                gstart(j + NBUF + k, k)
            return carry

        lax.fori_loop(0, rpt_e // NBUF - 1, body, 0)
        jlast = rpt_e - NBUF
        for k in range(NBUF):
            gwait(jlast + k, k)
            sstart(jlast + k, k)
        for k in range(NBUF):
            swait(jlast + k, k)
        plsc.subcore_barrier()
        pltpu.sync_copy(s_sh.at[rows], out_hbm.at[c, rows])

    a_flat = a.reshape(2 * NPAD, HALF)
    return k(a_flat, srcx, dst2d, z2d)


def _split(a):
    return jnp.stack([a[:, :HALF], a[:, HALF:]], axis=0).astype(jnp.bfloat16)


def _unsplit(s_ref, a_ref):
    f = jnp.float32
    return jnp.concatenate(
        [s_ref[0].astype(f) + a_ref[0].astype(f),
         s_ref[1].astype(f) + a_ref[1].astype(f)], axis=1)


def _inv_of(p_ref):
    return lax.rsqrt(p_ref[0, 0, :] + p_ref[1, 0, :] + 1.0)


def _scale_matmul(x, w1, parts):
    def body(x_ref, w_ref, p_ref, o_ref):
        inv = _inv_of(p_ref)
        h = jnp.dot(x_ref[...], w_ref[...], preferred_element_type=jnp.float32)
        o_ref[...] = _split(h * inv[:, None])

    return pl.pallas_call(
        body,
        grid=(NB,),
        in_specs=[
            pl.BlockSpec((BLK, D), lambda i: (i, 0)),
            pl.BlockSpec((D, D), lambda i: (0, 0)),
            pl.BlockSpec((2, 1, BLK), lambda i: (0, 0, i)),
        ],
        out_specs=pl.BlockSpec((2, BLK, HALF), lambda i: (0, i, 0)),
        out_shape=jax.ShapeDtypeStruct((2, NPAD, HALF), jnp.bfloat16),
    )(x, w1, parts)


def _conv_finish_matmul(s1, a1, parts, b1, w2):
    def body(s_ref, a_ref, p_ref, b_ref, w_ref, o_ref):
        inv = _inv_of(p_ref)
        x2 = jnp.maximum(_unsplit(s_ref, a_ref) * inv[:, None] + b_ref[...][None, :], 0.0)
        h = jnp.dot(x2, w_ref[...], preferred_element_type=jnp.float32)
        o_ref[...] = _split(h * inv[:, None])

    return pl.pallas_call(
        body,
        grid=(NB,),
        in_specs=[
            pl.BlockSpec((2, BLK, HALF), lambda i: (0, i, 0)),
            pl.BlockSpec((2, BLK, HALF), lambda i: (0, i, 0)),
            pl.BlockSpec((2, 1, BLK), lambda i: (0, 0, i)),
            pl.BlockSpec((D,), lambda i: (0,)),
            pl.BlockSpec((D, D), lambda i: (0, 0)),
        ],
        out_specs=pl.BlockSpec((2, BLK, HALF), lambda i: (0, i, 0)),
        out_shape=jax.ShapeDtypeStruct((2, NPAD, HALF), jnp.bfloat16),
    )(s1, a1, parts, b1, w2)


def _pool_lstm_heads(s2, a2, parts, b2, batchf, w_ih, b_ih, b_hh, wm_p, bm_p, wv_p, bv_p, C):
    def body(s_ref, a_ref, p_ref, b_ref, bf_ref, wih_ref, bih_ref, bhh_ref,
             wm_ref, bm_ref, wv_ref, bv_ref, mean_ref, lv_ref, sums, counts):
        i = pl.program_id(0)

        @pl.when(i == 0)
        def _():
            sums[...] = jnp.zeros_like(sums)
            counts[...] = jnp.zeros_like(counts)

        inv = _inv_of(p_ref)
        hb = jnp.maximum(_unsplit(s_ref, a_ref) * inv[:, None] + b_ref[...][None, :], 0.0)
        gid = lax.broadcasted_iota(jnp.int32, (G, BLK), 0).astype(jnp.float32)
        m = (gid == bf_ref[...][None, :]).astype(jnp.float32)
        sums[...] += jnp.dot(m, hb, preferred_element_type=jnp.float32)
        counts[...] += jnp.sum(m, axis=1)[None, :]

        @pl.when(i == NB - 1)
        def _():
            cnt = jnp.maximum(counts[0, :], 1.0)
            pooled = sums[...] / cnt[:, None]
            dn = (((1,), (1,)), ((), ()))
            gates = (lax.dot_general(pooled, wih_ref[...], dn,
                                     preferred_element_type=jnp.float32)
                     + bih_ref[...][None, :] + bhh_ref[...][None, :])
            i_g = jax.nn.sigmoid(gates[:, 0:G])
            g_g = jnp.tanh(gates[:, 2 * G:3 * G])
            o_g = jax.nn.sigmoid(gates[:, 3 * G:4 * G])
            h1 = o_g * jnp.tanh(i_g * g_g)
            mean = (lax.dot_general(h1, wm_ref[...], dn,
                                    preferred_element_type=jnp.float32)
                    + bm_ref[...][None, :])
            mean_ref[...] = mean[:, :mean_ref.shape[1]]
            lv = (lax.dot_general(h1, wv_ref[...], dn,
                                  preferred_element_type=jnp.float32)
                  + bv_ref[...][None, :])
            lv = jnp.maximum(lv, 0.0) + jnp.log1p(jnp.exp(-jnp.abs(lv)))
            lv_ref[...] = lv[:, :lv_ref.shape[1]]

    full = lambda shp: pl.BlockSpec(shp, lambda i: tuple(0 for _ in shp))
    return pl.pallas_call(
        body,
        grid=(NB,),
        in_specs=[
            pl.BlockSpec((2, BLK, HALF), lambda i: (0, i, 0)),
            pl.BlockSpec((2, BLK, HALF), lambda i: (0, i, 0)),
            pl.BlockSpec((2, 1, BLK), lambda i: (0, 0, i)),
            full((D,)),
            pl.BlockSpec((BLK,), lambda i: (i,)),
            full((4 * G, D)),
            full((4 * G,)),
            full((4 * G,)),
            full((G, G)),
            full((G,)),
            full((G, G)),
            full((G,)),
        ],
        out_specs=[full((G, C)), full((G, C))],
        out_shape=[
            jax.ShapeDtypeStruct((G, C), jnp.float32),
            jax.ShapeDtypeStruct((G, C), jnp.float32),
        ],
        scratch_shapes=[
            pltpu.VMEM((G, G), jnp.float32),
            pltpu.VMEM((1, G), jnp.float32),
        ],
    )(s2, a2, parts, b2, batchf, w_ih, b_ih, b_hh, wm_p, bm_p, wv_p, bv_p)


def kernel(x, edge_index, batch, W1, b1, W2, b2, W_ih, W_hh, b_ih, b_hh, Wm, bm, Wv, bv):
    C = Wm.shape[0]
    f32 = jnp.float32

    x_p = jnp.pad(x, ((0, NPAD - N), (0, 0)))
    pad_idx = (N + jnp.arange(EPAD - E, dtype=jnp.int32) % (NPAD - N))
    src2d = jnp.concatenate([edge_index[0], pad_idx]).reshape(NROW, CH)
    dst2d = jnp.concatenate([edge_index[1], pad_idx]).reshape(NROW, CH)
    srcx = jnp.stack([src2d, src2d + NPAD], axis=0)
    batchf = jnp.pad(batch, (0, NPAD - N), constant_values=-1).astype(f32)
    z1d = jnp.zeros((NPAD,), f32)
    z2d = jnp.zeros((NPAD, HALF), jnp.bfloat16)
    wm_p = jnp.zeros((G, G), f32).at[:C].set(Wm)
    bm_p = jnp.zeros((G,), f32).at[:C].set(bm)
    wv_p = jnp.zeros((G, G), f32).at[:C].set(Wv)
    bv_p = jnp.zeros((G,), f32).at[:C].set(bv)

    parts = _degree_parts(dst2d, z1d)
    a1 = _scale_matmul(x_p, W1, parts)
    s1 = _edge_scatter(a1, srcx, dst2d, z2d)
    a2 = _conv_finish_matmul(s1, a1, parts, b1, W2)
    s2 = _edge_scatter(a2, srcx, dst2d, z2d)
    mean_f, lv_f = _pool_lstm_heads(
        s2, a2, parts, b2, batchf, W_ih, b_ih, b_hh, wm_p, bm_p, wv_p, bv_p, C)
    return (mean_f, lv_f)


# speedup vs baseline: 1.0204x; 1.0204x over previous
import functools

import jax
import jax.numpy as jnp
from jax import lax
from jax.experimental import pallas as pl
from jax.experimental.pallas import tpu as pltpu
from jax.experimental.pallas import tpu_sc as plsc

N = 10000
E = 320000
D = 128
G = 128
NPAD = 10240
EPAD = 327680
HALF = 64
NS = 16
RPT = NPAD // NS
CH = 128
NROW = EPAD // CH
BLK = 5120
NB = NPAD // BLK

_mesh = lambda: plsc.VectorSubcoreMesh(core_axis_name="c", subcore_axis_name="s")
_sc_params = pltpu.CompilerParams(use_tc_tiling_on_sc=False)


def _degree_parts(dst2d, z1d):
    rpt_e = NROW // (2 * NS)

    @functools.partial(
        pl.kernel,
        out_type=jax.ShapeDtypeStruct((2, 1, NPAD), jnp.float32),
        mesh=_mesh(),
        compiler_params=_sc_params,
        scratch_types=[
            pltpu.VMEM((rpt_e, CH), jnp.int32),
            pltpu.VMEM((CH,), jnp.float32),
            pltpu.VMEM_SHARED((NPAD,), jnp.float32),
            pltpu.SemaphoreType.DMA,
        ],
    )
    def k(dst_hbm, z_hbm, out_hbm, dst_v, ones_v, hist_sh, sem):
        c = lax.axis_index("c")
        s = lax.axis_index("s")
        w = c * NS + s
        pltpu.sync_copy(z_hbm.at[pl.ds(s * RPT, RPT)], hist_sh.at[pl.ds(s * RPT, RPT)])
        pltpu.sync_copy(dst_hbm.at[pl.ds(w * rpt_e, rpt_e)], dst_v)
        for kk in range(CH // 16):
            ones_v[pl.ds(kk * 16, 16)] = jnp.ones((16,), jnp.float32)
        plsc.subcore_barrier()

        def body(i, carry):
            j = 8 * i
            for k in range(8):
                pltpu.async_copy(ones_v, hist_sh.at[dst_v.at[j + k]], sem, add=True)
            for k in range(8):
                pltpu.make_async_copy(ones_v, hist_sh.at[dst_v.at[j + k]], sem).wait()
            return carry

        lax.fori_loop(0, rpt_e // 8, body, 0)
        plsc.subcore_barrier()
        pltpu.sync_copy(hist_sh.at[pl.ds(s * RPT, RPT)],
                        out_hbm.at[c, 0, pl.ds(s * RPT, RPT)])

    return k(dst2d, z1d)


def _edge_scatter(a, srcx, dst2d, z2d):
    rpt_e = NROW // NS

    @functools.partial(
        pl.kernel,
        out_type=jax.ShapeDtypeStruct((2, NPAD, HALF), jnp.bfloat16),
        mesh=_mesh(),
        compiler_params=_sc_params,
        scratch_types=[
            pltpu.VMEM((rpt_e, CH), jnp.int32),
            pltpu.VMEM((rpt_e, CH), jnp.int32),
            [pltpu.VMEM((CH, HALF), jnp.bfloat16) for _ in range(8)],
            pltpu.VMEM_SHARED((NPAD, HALF), jnp.bfloat16),
            [pltpu.SemaphoreType.DMA for _ in range(8)],
            [pltpu.SemaphoreType.DMA for _ in range(8)],
            pltpu.SemaphoreType.DMA,
        ],
    )
    def k(a_hbm, src_hbm, dst_hbm, z_hbm, out_hbm,
          src_v, dst_v, msgs, s_sh, gsems, ssems, psem):
        NBUF = 8
        c = lax.axis_index("c")
        s = lax.axis_index("s")
        rows = pl.ds(s * RPT, RPT)
        pltpu.async_copy(src_hbm.at[c, pl.ds(s * rpt_e, rpt_e)], src_v, psem)
        pltpu.async_copy(dst_hbm.at[pl.ds(s * rpt_e, rpt_e)], dst_v, psem)
        pltpu.async_copy(z_hbm.at[rows], s_sh.at[rows], psem)
        pltpu.make_async_copy(src_hbm.at[c, pl.ds(s * rpt_e, rpt_e)], src_v, psem).wait()
        pltpu.make_async_copy(dst_hbm.at[pl.ds(s * rpt_e, rpt_e)], dst_v, psem).wait()
        pltpu.make_async_copy(z_hbm.at[rows], s_sh.at[rows], psem).wait()
        plsc.subcore_barrier()

        def gstart(j, k):
            pltpu.async_copy(a_hbm.at[src_v.at[j]], msgs[k], gsems[k])

        def gwait(j, k):
            pltpu.make_async_copy(a_hbm.at[src_v.at[j]], msgs[k], gsems[k]).wait()

        def sstart(j, k):
            pltpu.async_copy(msgs[k], s_sh.at[dst_v.at[j]], ssems[k], add=True)

        def swait(j, k):
            pltpu.make_async_copy(msgs[k], s_sh.at[dst_v.at[j]], ssems[k]).wait()

        for k in range(NBUF):
            gstart(k, k)

        def body(i, carry):
            j = NBUF * i
            for k in range(NBUF):
                gwait(j + k, k)
                sstart(j + k, k)
            for k in range(NBUF):
                swait(j + k, k)
                gstart(j + NBUF + k, k)
            return carry

        lax.fori_loop(0, rpt_e // NBUF - 1, body, 0)
        jlast = rpt_e - NBUF
        for k in range(NBUF):
            gwait(jlast + k, k)
            sstart(jlast + k, k)
        for k in range(NBUF):
            swait(jlast + k, k)
        plsc.subcore_barrier()
        pltpu.sync_copy(s_sh.at[rows], out_hbm.at[c, rows])

    a_flat = a.reshape(2 * NPAD, HALF)
    return k(a_flat, srcx, dst2d, z2d)


def _split(a):
    return jnp.stack([a[:, :HALF], a[:, HALF:]], axis=0).astype(jnp.bfloat16)


def _unsplit(s_ref, a_ref):
    f = jnp.float32
    return jnp.concatenate(
        [s_ref[0].astype(f) + a_ref[0].astype(f),
         s_ref[1].astype(f) + a_ref[1].astype(f)], axis=1)


def _inv_of(p_ref):
    return lax.rsqrt(p_ref[0, 0, :] + p_ref[1, 0, :] + 1.0)


def _scale_matmul(x, w1, parts):
    def body(x_ref, w_ref, p_ref, o_ref):
        inv = _inv_of(p_ref)
        h = jnp.dot(x_ref[...], w_ref[...], preferred_element_type=jnp.float32)
        o_ref[...] = _split(h * inv[:, None])

    return pl.pallas_call(
        body,
        grid=(NB,),
        in_specs=[
            pl.BlockSpec((BLK, D), lambda i: (i, 0)),
            pl.BlockSpec((D, D), lambda i: (0, 0)),
            pl.BlockSpec((2, 1, BLK), lambda i: (0, 0, i)),
        ],
        out_specs=pl.BlockSpec((2, BLK, HALF), lambda i: (0, i, 0)),
        out_shape=jax.ShapeDtypeStruct((2, NPAD, HALF), jnp.bfloat16),
    )(x, w1, parts)


def _conv_finish_matmul(s1, a1, parts, b1, w2):
    def body(s_ref, a_ref, p_ref, b_ref, w_ref, o_ref):
        inv = _inv_of(p_ref)
        x2 = jnp.maximum(_unsplit(s_ref, a_ref) * inv[:, None] + b_ref[...][None, :], 0.0)
        h = jnp.dot(x2, w_ref[...], preferred_element_type=jnp.float32)
        o_ref[...] = _split(h * inv[:, None])

    return pl.pallas_call(
        body,
        grid=(NB,),
        in_specs=[
            pl.BlockSpec((2, BLK, HALF), lambda i: (0, i, 0)),
            pl.BlockSpec((2, BLK, HALF), lambda i: (0, i, 0)),
            pl.BlockSpec((2, 1, BLK), lambda i: (0, 0, i)),
            pl.BlockSpec((D,), lambda i: (0,)),
            pl.BlockSpec((D, D), lambda i: (0, 0)),
        ],
        out_specs=pl.BlockSpec((2, BLK, HALF), lambda i: (0, i, 0)),
        out_shape=jax.ShapeDtypeStruct((2, NPAD, HALF), jnp.bfloat16),
    )(s1, a1, parts, b1, w2)


def _pool_lstm_heads(s2, a2, parts, b2, batchf, w_ih, b_ih, b_hh, wm_p, bm_p, wv_p, bv_p, C):
    def body(s_ref, a_ref, p_ref, b_ref, bf_ref, wih_ref, bih_ref, bhh_ref,
             wm_ref, bm_ref, wv_ref, bv_ref, mean_ref, lv_ref, sums, counts):
        i = pl.program_id(0)

        @pl.when(i == 0)
        def _():
            sums[...] = jnp.zeros_like(sums)
            counts[...] = jnp.zeros_like(counts)

        inv = _inv_of(p_ref)
        hb = jnp.maximum(_unsplit(s_ref, a_ref) * inv[:, None] + b_ref[...][None, :], 0.0)
        gid = lax.broadcasted_iota(jnp.int32, (G, BLK), 0).astype(jnp.float32)
        m = (gid == bf_ref[...][None, :]).astype(jnp.float32)
        sums[...] += jnp.dot(m, hb, preferred_element_type=jnp.float32)
        counts[...] += jnp.sum(m, axis=1)[None, :]

        @pl.when(i == NB - 1)
        def _():
            cnt = jnp.maximum(counts[0, :], 1.0)
            pooled = sums[...] / cnt[:, None]
            dn = (((1,), (1,)), ((), ()))
            gates = (lax.dot_general(pooled, wih_ref[...], dn,
                                     preferred_element_type=jnp.float32)
                     + bih_ref[...][None, :] + bhh_ref[...][None, :])
            i_g = jax.nn.sigmoid(gates[:, 0:G])
            g_g = jnp.tanh(gates[:, 2 * G:3 * G])
            o_g = jax.nn.sigmoid(gates[:, 3 * G:4 * G])
            h1 = o_g * jnp.tanh(i_g * g_g)
            mean = (lax.dot_general(h1, wm_ref[...], dn,
                                    preferred_element_type=jnp.float32)
                    + bm_ref[...][None, :])
            mean_ref[...] = mean[:, :mean_ref.shape[1]]
            lv = (lax.dot_general(h1, wv_ref[...], dn,
                                  preferred_element_type=jnp.float32)
                  + bv_ref[...][None, :])
            lv = jnp.maximum(lv, 0.0) + jnp.log1p(jnp.exp(-jnp.abs(lv)))
            lv_ref[...] = lv[:, :lv_ref.shape[1]]

    full = lambda shp: pl.BlockSpec(shp, lambda i: tuple(0 for _ in shp))
    return pl.pallas_call(
        body,
        grid=(NB,),
        in_specs=[
            pl.BlockSpec((2, BLK, HALF), lambda i: (0, i, 0)),
            pl.BlockSpec((2, BLK, HALF), lambda i: (0, i, 0)),
            pl.BlockSpec((2, 1, BLK), lambda i: (0, 0, i)),
            full((D,)),
            pl.BlockSpec((BLK,), lambda i: (i,)),
            full((4 * G, D)),
            full((4 * G,)),
            full((4 * G,)),
            full((G, G)),
            full((G,)),
            full((G, G)),
            full((G,)),
        ],
        out_specs=[full((G, C)), full((G, C))],
        out_shape=[
            jax.ShapeDtypeStruct((G, C), jnp.float32),
            jax.ShapeDtypeStruct((G, C), jnp.float32),
        ],
        scratch_shapes=[
            pltpu.VMEM((G, G), jnp.float32),
            pltpu.VMEM((1, G), jnp.float32),
        ],
    )(s2, a2, parts, b2, batchf, w_ih, b_ih, b_hh, wm_p, bm_p, wv_p, bv_p)


def kernel(x, edge_index, batch, W1, b1, W2, b2, W_ih, W_hh, b_ih, b_hh, Wm, bm, Wv, bv):
    C = Wm.shape[0]
    f32 = jnp.float32

    x_p = jnp.pad(x, ((0, NPAD - N), (0, 0)))
    pad_idx = (N + jnp.arange(EPAD - E, dtype=jnp.int32) % (NPAD - N))
    src2d = jnp.concatenate([edge_index[0], pad_idx]).reshape(NROW, CH)
    dst2d = jnp.concatenate([edge_index[1], pad_idx]).reshape(NROW, CH)
    srcx = jnp.stack([src2d, src2d + NPAD], axis=0)
    batchf = jnp.pad(batch, (0, NPAD - N), constant_values=-1).astype(f32)
    z1d = jnp.zeros((NPAD,), f32)
    z2d = jnp.zeros((NPAD, HALF), jnp.bfloat16)
    wm_p = jnp.zeros((G, G), f32).at[:C].set(Wm)
    bm_p = jnp.zeros((G,), f32).at[:C].set(bm)
    wv_p = jnp.zeros((G, G), f32).at[:C].set(Wv)
    bv_p = jnp.zeros((G,), f32).at[:C].set(bv)

    parts = _degree_parts(dst2d, z1d)
    a1 = _scale_matmul(x_p, W1, parts)
    s1 = _edge_scatter(a1, srcx, dst2d, z2d)
    a2 = _conv_finish_matmul(s1, a1, parts, b1, W2)
    s2 = _edge_scatter(a2, srcx, dst2d, z2d)
    mean_f, lv_f = _pool_lstm_heads(
        s2, a2, parts, b2, batchf, W_ih, b_ih, b_hh, wm_p, bm_p, wv_p, bv_p, C)
    return (mean_f, lv_f)
